# baseline (speedup 1.0000x reference)
import functools

import jax
import jax.numpy as jnp
from jax import lax
from jax.experimental import pallas as pl
from jax.experimental.pallas import tpu as pltpu
from jax.experimental.pallas import tpu_sc as plsc

B = 16384
F = 26
V = 100000
D = 16

NC = 2
NS = 16
NW = NC * NS

ROWS = B * F
RPW = ROWS // NW
CH = 1664
NCH = RPW // CH

_mesh = plsc.VectorSubcoreMesh(core_axis_name="c", subcore_axis_name="s")


@functools.partial(
    pl.kernel,
    mesh=_mesh,
    out_type=jax.ShapeDtypeStruct((ROWS, D), jnp.float32),
    compiler_params=pltpu.CompilerParams(use_tc_tiling_on_sc=False),
    scratch_types=[
        pltpu.VMEM((RPW,), jnp.int32),
        pltpu.VMEM((2, CH, D), jnp.float32),
        pltpu.SemaphoreType.DMA,
        pltpu.SemaphoreType.DMA,
    ],
)
def _gather_kernel(idx_hbm, table_hbm, out_hbm, idx_v, bufs, sem0, sem1):
    wid = lax.axis_index("s") * NC + lax.axis_index("c")
    base = wid * RPW

    pltpu.sync_copy(idx_hbm.at[pl.ds(base, RPW)], idx_v)

    sems = (sem0, sem1)
    handles = [None, None]

    def start_gather(j):
        handles[j % 2] = pltpu.async_copy(
            table_hbm.at[idx_v.at[pl.ds(j * CH, CH)]],
            bufs.at[j % 2],
            sems[j % 2],
        )

    start_gather(0)
    for j in range(1, NCH):
        start_gather(j)
        handles[(j - 1) % 2].wait()
        pltpu.sync_copy(
            bufs.at[(j - 1) % 2],
            out_hbm.at[pl.ds(base + (j - 1) * CH, CH)],
        )
    handles[(NCH - 1) % 2].wait()
    pltpu.sync_copy(
        bufs.at[(NCH - 1) % 2],
        out_hbm.at[pl.ds(base + (NCH - 1) * CH, CH)],
    )


def kernel(node_feature, tables):
    flat_idx = (
        node_feature.astype(jnp.int32)
        + (jnp.arange(F, dtype=jnp.int32) * V)[None, :]
    ).reshape(ROWS)
    table2d = tables.reshape(F * V, D)
    out = _gather_kernel(flat_idx, table2d)
    return out.reshape(B, F * D)
